# Initial kernel scaffold; baseline (speedup 1.0000x reference)
import jax
import jax.numpy as jnp
from jax.experimental import pallas as pl


def kernel(encoder_logits, u_noise):
    raise NotImplementedError("write your pallas kernel here")


# speedup vs baseline: 6.5713x; 6.5713x over previous
import functools

import jax
import jax.numpy as jnp
from jax.experimental import pallas as pl

B, V, C = 64, 32, 64
R = B * V

_NEG = -1e30


def _shl(x, k, pad):
    return jnp.concatenate(
        [x[:, k:], jnp.full((x.shape[0], k), pad, x.dtype)], axis=1)


def _shr(x, k, pad):
    return jnp.concatenate(
        [jnp.full((x.shape[0], k), pad, x.dtype), x[:, : x.shape[1] - k]],
        axis=1)


def _vae_kernel(x_ref, u_ref, iw_ref, s_ref):
    x = x_ref[:]
    u = u_ref[:]
    lane = jax.lax.broadcasted_iota(jnp.int32, (R, C), 1)
    valid = lane < C - 1

    m_row = jnp.max(x, axis=1, keepdims=True)
    e = jnp.exp(x - m_row)
    t = e
    for k in (1, 2, 4, 8, 16, 32):
        t = t + _shl(t, k, 0.0)
    s = _shl(t, 1, 0.0)
    m = _shl(x, 1, _NEG)
    for k in (1, 2, 4, 8, 16, 32):
        m = jnp.maximum(m, _shl(m, k, _NEG))
    d = jnp.where(valid, m_row - m, 0.0)
    denom = m + jnp.log(s * jnp.exp(d))
    sb = jnp.where(valid, x - denom, 0.0)

    sg = jax.nn.sigmoid(sb)
    sg_abs = jnp.maximum(sg, 1.0 - sg)
    cond = sb >= -1e-5
    safe_den = jnp.where(cond, 1.0, 1.0 - 2.0 * sg)
    bzm = jnp.where(cond, 0.0, (1.0 - sg) ** 2 / safe_den)
    p = bzm
    for k in (1, 2, 4, 8, 16, 32):
        p = p * _shr(p, k, 1.0)
    iw_ref[:] = _shr(p, 1, 1.0) * jnp.where(valid, sg_abs, 1.0)

    hit = (u < sg) & valid
    s_ref[:] = jnp.min(jnp.where(hit, lane, C - 1), axis=1, keepdims=True)


@jax.jit
def kernel(encoder_logits, u_noise):
    x = encoder_logits.reshape(R, C)
    u = u_noise.reshape(R, C)
    iw, samp = pl.pallas_call(
        _vae_kernel,
        out_shape=(
            jax.ShapeDtypeStruct((R, C), jnp.float32),
            jax.ShapeDtypeStruct((R, 1), jnp.int32),
        ),
    )(x, u)
    return iw.reshape(B, V, C), samp.reshape(B, V)
